# baseline (speedup 1.0000x reference)
import functools

import jax
import jax.numpy as jnp
from jax import lax
from jax.experimental import pallas as pl
from jax.experimental.pallas import tpu as pltpu
from jax.experimental.pallas import tpu_sc as plsc

B = 16384
F = 26
V = 1000000
E = 16
D = F * E
H1 = 1024
H2 = 512

NUM_WORKERS = 32

SLAB = 512
NSLAB = 61 * NUM_WORKERS
SLAB_REM = V - NSLAB * SLAB
REM0 = NSLAB * SLAB
REM_A = 512
REM_B = SLAB_REM - REM_A
KPT = NSLAB // NUM_WORKERS

IDX_PER_TILE = (B * F) // NUM_WORKERS
CHUNK = 3328
NCHUNK = IDX_PER_TILE // CHUNK
DCHUNK = 6656
NDCHUNK = IDX_PER_TILE // DCHUNK


def _transpose_slab(tbuf, obuf, iota16, n_ch):
    def ch_body(ch, c):
        for e in range(E):
            vals = tbuf[e, pl.ds(ch * 16, 16)]
            idxv = iota16 * 16 + (ch * 256 + e)
            plsc.store_scatter(obuf, [idxv], vals)
        return c
    lax.fori_loop(0, n_ch, ch_body, 0)


def _colsum_slab(tbuf, csbuf, n_ch):
    def ch_body(ch, c):
        acc = tbuf[0, pl.ds(ch * 16, 16)]
        for e in range(1, E):
            acc = acc + tbuf[e, pl.ds(ch * 16, 16)]
        csbuf[pl.ds(ch * 16, 16)] = acc
        return c
    lax.fori_loop(0, n_ch, ch_body, 0)


@functools.lru_cache(maxsize=1)
def _make_sc_format():
    mesh = plsc.VectorSubcoreMesh(core_axis_name="c", subcore_axis_name="s")

    @functools.partial(
        pl.kernel,
        mesh=mesh,
        out_type=[
            jax.ShapeDtypeStruct((V * E,), jnp.float32),
            jax.ShapeDtypeStruct((V,), jnp.float32),
        ],
        scratch_types=[
            pltpu.VMEM((E, SLAB), jnp.float32),
            pltpu.VMEM((E, SLAB), jnp.float32),
            pltpu.VMEM((SLAB * E,), jnp.float32),
            pltpu.VMEM((SLAB * E,), jnp.float32),
            pltpu.VMEM((SLAB,), jnp.float32),
            pltpu.VMEM((SLAB,), jnp.float32),
            pltpu.SemaphoreType.DMA,
            pltpu.SemaphoreType.DMA,
            pltpu.SemaphoreType.DMA,
            pltpu.SemaphoreType.DMA,
        ],
        compiler_params=pltpu.CompilerParams(
            use_tc_tiling_on_sc=True, needs_layout_passes=False),
    )
    def _sc_format(wideT, deepT, tail_deep, tail_cs, ddense, csum,
                   tbuf0, tbuf1, obuf0, obuf1, csbuf0, csbuf1,
                   sin0, sin1, sout0, sout1):
        wid = lax.axis_index("s") * 2 + lax.axis_index("c")
        iota16 = lax.iota(jnp.int32, 16)
        tb = (tbuf0, tbuf1)
        ob = (obuf0, obuf1)
        cb = (csbuf0, csbuf1)
        sin = (sin0, sin1)
        sout = (sout0, sout1)

        def col0(k):
            return (wid + NUM_WORKERS * k) * SLAB

        def wait_in(src, p):
            pltpu.make_async_copy(
                src.at[:, pl.ds(0, SLAB)], tb[p], sin[p]).wait()

        def deep_step(k):
            wait_in(deepT, 0)

            @pl.when(k >= 1)
            def _():
                pltpu.make_async_copy(
                    ob[0], ddense.at[pl.ds(0, SLAB * E)], sout[0]).wait()
            _transpose_slab(tb[0], ob[0], iota16, SLAB // 16)
            pltpu.async_copy(
                ob[0], ddense.at[pl.ds(col0(k) * E, SLAB * E)], sout[0])

            @pl.when(k < 60)
            def _():
                pltpu.async_copy(
                    deepT.at[:, pl.ds(col0(k + 1), SLAB)], tb[0], sin[0])

        def wide_step(k):
            wait_in(wideT, 1)

            @pl.when(k >= 1)
            def _():
                pltpu.make_async_copy(
                    cb[1], csum.at[pl.ds(0, SLAB)], sout[1]).wait()
            _colsum_slab(tb[1], cb[1], SLAB // 16)
            pltpu.async_copy(
                cb[1], csum.at[pl.ds(col0(k), SLAB)], sout[1])

            @pl.when(k < 60)
            def _():
                pltpu.async_copy(
                    wideT.at[:, pl.ds(col0(k + 1), SLAB)], tb[1], sin[1])

        pltpu.async_copy(deepT.at[:, pl.ds(col0(0), SLAB)], tb[0], sin[0])
        pltpu.async_copy(wideT.at[:, pl.ds(col0(0), SLAB)], tb[1], sin[1])

        def body(k, c):
            deep_step(k)
            wide_step(k)
            return c

        lax.fori_loop(0, 61, body, 0)
        pltpu.make_async_copy(
            ob[0], ddense.at[pl.ds(0, SLAB * E)], sout[0]).wait()
        pltpu.make_async_copy(
            cb[1], csum.at[pl.ds(0, SLAB)], sout[1]).wait()

        @pl.when(wid == 28)
        def _():
            pltpu.sync_copy(deepT.at[:, pl.ds(REM0, REM_A)],
                            tbuf0.at[:, pl.ds(0, REM_A)])
            _transpose_slab(tbuf0, obuf0, iota16, REM_A // 16)
            pltpu.sync_copy(obuf0.at[pl.ds(0, REM_A * E)],
                            ddense.at[pl.ds(REM0 * E, REM_A * E)])

        @pl.when(wid == 29)
        def _():
            pltpu.sync_copy(tail_deep, obuf1.at[pl.ds(0, REM_B * E)])
            pltpu.sync_copy(obuf1.at[pl.ds(0, REM_B * E)],
                            ddense.at[pl.ds((REM0 + REM_A) * E, REM_B * E)])

        @pl.when(wid == 30)
        def _():
            pltpu.sync_copy(wideT.at[:, pl.ds(REM0, REM_A)],
                            tbuf1.at[:, pl.ds(0, REM_A)])
            _colsum_slab(tbuf1, csbuf0, REM_A // 16)
            pltpu.sync_copy(csbuf0.at[pl.ds(0, REM_A)],
                            csum.at[pl.ds(REM0, REM_A)])

        @pl.when(wid == 31)
        def _():
            pltpu.sync_copy(tail_cs, csbuf1.at[pl.ds(0, REM_B)])
            pltpu.sync_copy(csbuf1.at[pl.ds(0, REM_B)],
                            csum.at[pl.ds(REM0 + REM_A, REM_B)])

    return _sc_format


@functools.lru_cache(maxsize=1)
def _make_sc_gather_deep():
    mesh = plsc.VectorSubcoreMesh(core_axis_name="c", subcore_axis_name="s")

    @functools.partial(
        pl.kernel,
        mesh=mesh,
        out_type=jax.ShapeDtypeStruct((B * F, E), jnp.float32),
        scratch_types=[
            pltpu.VMEM((DCHUNK,), jnp.int32),
            pltpu.VMEM((DCHUNK, E), jnp.float32),
            pltpu.SemaphoreType.DMA,
            pltpu.SemaphoreType.DMA,
        ],
        compiler_params=pltpu.CompilerParams(
            use_tc_tiling_on_sc=False, needs_layout_passes=False),
    )
    def _sc_gather_deep(idx_hbm, deep_hbm, deep_out, idx_v, drows,
                        sem_g, sem_o):
        wid = lax.axis_index("s") * 2 + lax.axis_index("c")
        base = wid * IDX_PER_TILE
        pltpu.sync_copy(idx_hbm.at[pl.ds(base, DCHUNK)], idx_v)
        pltpu.async_copy(deep_hbm.at[idx_v], drows, sem_g).wait()
        pltpu.async_copy(drows, deep_out.at[pl.ds(base, DCHUNK)], sem_o)
        pltpu.sync_copy(idx_hbm.at[pl.ds(base + DCHUNK, DCHUNK)], idx_v)
        pltpu.make_async_copy(
            drows, deep_out.at[pl.ds(0, DCHUNK)], sem_o).wait()
        pltpu.async_copy(deep_hbm.at[idx_v], drows, sem_g).wait()
        pltpu.sync_copy(drows, deep_out.at[pl.ds(base + DCHUNK, DCHUNK)])

    return _sc_gather_deep


@functools.lru_cache(maxsize=1)
def _make_sc_gather_wide():
    mesh = plsc.VectorSubcoreMesh(core_axis_name="c", subcore_axis_name="s")

    @functools.partial(
        pl.kernel,
        mesh=mesh,
        out_type=jax.ShapeDtypeStruct((B * F,), jnp.float32),
        scratch_types=[
            pltpu.VMEM((CHUNK,), jnp.int32),
            pltpu.VMEM((CHUNK,), jnp.int32),
            pltpu.VMEM((CHUNK, E), jnp.float32),
            pltpu.VMEM((CHUNK,), jnp.float32),
            pltpu.SemaphoreType.DMA,
        ],
        compiler_params=pltpu.CompilerParams(
            use_tc_tiling_on_sc=False, needs_layout_passes=False),
    )
    def _sc_gather_wide(idx_hbm, cs2d_hbm, wsum_out,
                        idx_v, widx_v, wrows, wv, sem_w):
        wid = lax.axis_index("s") * 2 + lax.axis_index("c")
        base = wid * IDX_PER_TILE
        iota16 = lax.iota(jnp.int32, 16)

        def chunk_body(ci, carry):
            off = base + ci * CHUNK
            pltpu.sync_copy(idx_hbm.at[pl.ds(off, CHUNK)], idx_v)

            def wi_body(t, c2):
                iv = idx_v[pl.ds(t * 16, 16)]
                widx_v[pl.ds(t * 16, 16)] = lax.shift_right_logical(iv, 4)
                return c2
            lax.fori_loop(0, CHUNK // 16, wi_body, 0)

            pltpu.async_copy(cs2d_hbm.at[widx_v], wrows, sem_w).wait()

            def ex_body(t, c2):
                rows = iota16 + t * 16
                cm = lax.bitwise_and(idx_v[pl.ds(t * 16, 16)], 15)
                wv[pl.ds(t * 16, 16)] = plsc.load_gather(wrows, [rows, cm])
                return c2
            lax.fori_loop(0, CHUNK // 16, ex_body, 0)
            pltpu.sync_copy(wv, wsum_out.at[pl.ds(off, CHUNK)])
            return carry

        lax.fori_loop(0, NCHUNK, chunk_body, 0)

    return _sc_gather_wide


def _stats_body(x_ref, s1_ref, s2_ref):
    i = pl.program_id(0)

    @pl.when(i == 0)
    def _():
        s1_ref[...] = jnp.zeros_like(s1_ref)
        s2_ref[...] = jnp.zeros_like(s2_ref)

    xb = x_ref[...]
    s1_ref[...] += jnp.sum(xb, axis=0, keepdims=True)
    s2_ref[...] += jnp.sum(xb * xb, axis=0, keepdims=True)


def _mlp_body(s1_ref, s2_ref, gamma_ref, beta_ref, w1_ref, b1_ref,
              w2_ref, b2_ref, w3_ref, b3_ref, x_ref, wide_ref, out_ref):
    mean = s1_ref[...] * (1.0 / B)
    var = s2_ref[...] * (1.0 / B) - mean * mean
    scale = gamma_ref[...] * lax.rsqrt(var + 1e-5)
    shift = beta_ref[...] - mean * scale
    xn = x_ref[...] * scale + shift
    h = jnp.dot(xn.astype(jnp.bfloat16), w1_ref[...],
                preferred_element_type=jnp.float32)
    h = jnp.maximum(h + b1_ref[...], 0.0)
    h = jnp.dot(h.astype(jnp.bfloat16), w2_ref[...],
                preferred_element_type=jnp.float32)
    h = jnp.maximum(h + b2_ref[...], 0.0)
    d = jnp.dot(h.astype(jnp.bfloat16), w3_ref[...],
                preferred_element_type=jnp.float32)
    d = d + b3_ref[...]
    w = jnp.sum(wide_ref[...], axis=1, keepdims=True)
    out_ref[...] = jax.nn.sigmoid(d + w)


_STATS_BLK = 2048
_MLP_BLK = 2048


@jax.jit
def kernel(inputs, wide_table, deep_table, gamma, beta, W1, b1, W2, b2, W3, b3):
    idx_flat = inputs.reshape(B * F).astype(jnp.int32)

    tail_deep = deep_table[REM0 + REM_A:, :].reshape(REM_B * E)
    tail_cs = jnp.sum(wide_table[REM0 + REM_A:, :], axis=1)
    deep_dense, wide_colsum = _make_sc_format()(
        wide_table.T, deep_table.T, tail_deep, tail_cs)
    deep_rows = _make_sc_gather_deep()(idx_flat, deep_dense.reshape(V, E))
    wsum = _make_sc_gather_wide()(idx_flat, wide_colsum.reshape(V // E, E))
    x = deep_rows.reshape(B, D)
    wmat = wsum.reshape(B, F)

    s1, s2 = pl.pallas_call(
        _stats_body,
        grid=(B // _STATS_BLK,),
        in_specs=[pl.BlockSpec((_STATS_BLK, D), lambda i: (i, 0))],
        out_specs=[
            pl.BlockSpec((1, D), lambda i: (0, 0)),
            pl.BlockSpec((1, D), lambda i: (0, 0)),
        ],
        out_shape=[
            jax.ShapeDtypeStruct((1, D), jnp.float32),
            jax.ShapeDtypeStruct((1, D), jnp.float32),
        ],
    )(x)

    out = pl.pallas_call(
        _mlp_body,
        grid=(B // _MLP_BLK,),
        in_specs=[
            pl.BlockSpec((1, D), lambda i: (0, 0)),
            pl.BlockSpec((1, D), lambda i: (0, 0)),
            pl.BlockSpec((1, D), lambda i: (0, 0)),
            pl.BlockSpec((1, D), lambda i: (0, 0)),
            pl.BlockSpec((D, H1), lambda i: (0, 0)),
            pl.BlockSpec((1, H1), lambda i: (0, 0)),
            pl.BlockSpec((H1, H2), lambda i: (0, 0)),
            pl.BlockSpec((1, H2), lambda i: (0, 0)),
            pl.BlockSpec((H2, 1), lambda i: (0, 0)),
            pl.BlockSpec((1, 1), lambda i: (0, 0)),
            pl.BlockSpec((_MLP_BLK, D), lambda i: (i, 0)),
            pl.BlockSpec((_MLP_BLK, F), lambda i: (i, 0)),
        ],
        out_specs=pl.BlockSpec((_MLP_BLK, 1), lambda i: (i, 0)),
        out_shape=jax.ShapeDtypeStruct((B, 1), jnp.float32),
    )(
        s1, s2,
        gamma.reshape(1, D), beta.reshape(1, D),
        W1.astype(jnp.bfloat16), b1.reshape(1, H1),
        W2.astype(jnp.bfloat16), b2.reshape(1, H2),
        W3.astype(jnp.bfloat16), b3.reshape(1, 1),
        x, wmat,
    )
    return out


# speedup vs baseline: 1.1290x; 1.1290x over previous
import functools

import jax
import jax.numpy as jnp
from jax import lax
from jax.experimental import pallas as pl
from jax.experimental.pallas import tpu as pltpu
from jax.experimental.pallas import tpu_sc as plsc

B = 16384
F = 26
V = 1000000
E = 16
D = F * E
H1 = 1024
H2 = 512

NUM_WORKERS = 32

SLAB = 512
NSLAB = 61 * NUM_WORKERS
SLAB_REM = V - NSLAB * SLAB
REM0 = NSLAB * SLAB
REM_A = 512
REM_B = SLAB_REM - REM_A
KPT = NSLAB // NUM_WORKERS

IDX_PER_TILE = (B * F) // NUM_WORKERS
CHUNK = 3328
NCHUNK = IDX_PER_TILE // CHUNK
DCHUNK = 6656
NDCHUNK = IDX_PER_TILE // DCHUNK


def _transpose_slab(tbuf, obuf, iota16, n_ch):
    def ch_body(ch, c):
        for e in range(E):
            vals = tbuf[e, pl.ds(ch * 16, 16)]
            idxv = iota16 * 16 + (ch * 256 + e)
            plsc.store_scatter(obuf, [idxv], vals)
        return c
    lax.fori_loop(0, n_ch, ch_body, 0)


def _colsum_slab(tbuf, csbuf, n_ch):
    def ch_body(ch, c):
        acc = tbuf[0, pl.ds(ch * 16, 16)]
        for e in range(1, E):
            acc = acc + tbuf[e, pl.ds(ch * 16, 16)]
        csbuf[pl.ds(ch * 16, 16)] = acc
        return c
    lax.fori_loop(0, n_ch, ch_body, 0)


@functools.lru_cache(maxsize=1)
def _make_sc_format():
    mesh = plsc.VectorSubcoreMesh(core_axis_name="c", subcore_axis_name="s")

    @functools.partial(
        pl.kernel,
        mesh=mesh,
        out_type=jax.ShapeDtypeStruct((V * E,), jnp.float32),
        scratch_types=[
            pltpu.VMEM((E, SLAB), jnp.float32),
            pltpu.VMEM((E, SLAB), jnp.float32),
            pltpu.VMEM((SLAB * E,), jnp.float32),
            pltpu.VMEM((SLAB * E,), jnp.float32),
            pltpu.SemaphoreType.DMA,
            pltpu.SemaphoreType.DMA,
            pltpu.SemaphoreType.DMA,
            pltpu.SemaphoreType.DMA,
        ],
        compiler_params=pltpu.CompilerParams(
            use_tc_tiling_on_sc=True, needs_layout_passes=False),
    )
    def _sc_format(deepT, tail_deep, ddense,
                   tbuf0, tbuf1, obuf0, obuf1,
                   sin0, sin1, sout0, sout1):
        wid = lax.axis_index("s") * 2 + lax.axis_index("c")
        iota16 = lax.iota(jnp.int32, 16)
        tb = (tbuf0, tbuf1)
        ob = (obuf0, obuf1)
        sin = (sin0, sin1)
        sout = (sout0, sout1)

        def col0(k):
            return (wid + NUM_WORKERS * k) * SLAB

        def wait_in(p):
            pltpu.make_async_copy(
                deepT.at[:, pl.ds(0, SLAB)], tb[p], sin[p]).wait()

        def wait_out(p):
            pltpu.make_async_copy(
                ob[p], ddense.at[pl.ds(0, SLAB * E)], sout[p]).wait()

        def compute_and_out(k, p):
            c0 = col0(k)
            _transpose_slab(tb[p], ob[p], iota16, SLAB // 16)
            pltpu.async_copy(
                ob[p], ddense.at[pl.ds(c0 * E, SLAB * E)], sout[p])

        for p in range(2):
            pltpu.async_copy(deepT.at[:, pl.ds(col0(p), SLAB)],
                             tb[p], sin[p])

        def body2(k2, c):
            for p in range(2):
                k = 2 * k2 + p
                wait_in(p)

                @pl.when(k2 >= 1)
                def _():
                    wait_out(p)
                compute_and_out(k, p)
                if p == 0:
                    pltpu.async_copy(
                        deepT.at[:, pl.ds(col0(k + 2), SLAB)], tb[p], sin[p])
                else:
                    @pl.when(k2 < 29)
                    def _():
                        pltpu.async_copy(
                            deepT.at[:, pl.ds(col0(k + 2), SLAB)],
                            tb[p], sin[p])
            return c

        lax.fori_loop(0, 30, body2, 0)
        wait_in(0)
        wait_out(0)
        compute_and_out(60, 0)
        wait_out(0)
        wait_out(1)

        @pl.when(wid == 28)
        def _():
            pltpu.sync_copy(deepT.at[:, pl.ds(REM0, REM_A)],
                            tbuf0.at[:, pl.ds(0, REM_A)])
            _transpose_slab(tbuf0, obuf0, iota16, REM_A // 16)
            pltpu.sync_copy(obuf0.at[pl.ds(0, REM_A * E)],
                            ddense.at[pl.ds(REM0 * E, REM_A * E)])

        @pl.when(wid == 29)
        def _():
            pltpu.sync_copy(tail_deep, obuf1.at[pl.ds(0, REM_B * E)])
            pltpu.sync_copy(obuf1.at[pl.ds(0, REM_B * E)],
                            ddense.at[pl.ds((REM0 + REM_A) * E, REM_B * E)])

    return _sc_format


@functools.lru_cache(maxsize=1)
def _make_sc_gather_deep():
    mesh = plsc.VectorSubcoreMesh(core_axis_name="c", subcore_axis_name="s")

    @functools.partial(
        pl.kernel,
        mesh=mesh,
        out_type=jax.ShapeDtypeStruct((B * F, E), jnp.float32),
        scratch_types=[
            pltpu.VMEM((DCHUNK,), jnp.int32),
            pltpu.VMEM((DCHUNK, E), jnp.float32),
            pltpu.SemaphoreType.DMA,
            pltpu.SemaphoreType.DMA,
        ],
        compiler_params=pltpu.CompilerParams(
            use_tc_tiling_on_sc=False, needs_layout_passes=False),
    )
    def _sc_gather_deep(idx_hbm, deep_hbm, deep_out, idx_v, drows,
                        sem_g, sem_o):
        wid = lax.axis_index("s") * 2 + lax.axis_index("c")
        base = wid * IDX_PER_TILE
        pltpu.sync_copy(idx_hbm.at[pl.ds(base, DCHUNK)], idx_v)
        pltpu.async_copy(deep_hbm.at[idx_v], drows, sem_g).wait()
        pltpu.async_copy(drows, deep_out.at[pl.ds(base, DCHUNK)], sem_o)
        pltpu.sync_copy(idx_hbm.at[pl.ds(base + DCHUNK, DCHUNK)], idx_v)
        pltpu.make_async_copy(
            drows, deep_out.at[pl.ds(0, DCHUNK)], sem_o).wait()
        pltpu.async_copy(deep_hbm.at[idx_v], drows, sem_g).wait()
        pltpu.sync_copy(drows, deep_out.at[pl.ds(base + DCHUNK, DCHUNK)])

    return _sc_gather_deep


@functools.lru_cache(maxsize=1)
def _make_sc_gather_wide():
    mesh = plsc.VectorSubcoreMesh(core_axis_name="c", subcore_axis_name="s")

    @functools.partial(
        pl.kernel,
        mesh=mesh,
        out_type=jax.ShapeDtypeStruct((B * F,), jnp.float32),
        scratch_types=[
            pltpu.VMEM((CHUNK,), jnp.int32),
            pltpu.VMEM((CHUNK,), jnp.int32),
            pltpu.VMEM((CHUNK, E), jnp.float32),
            pltpu.VMEM((CHUNK,), jnp.float32),
            pltpu.SemaphoreType.DMA,
        ],
        compiler_params=pltpu.CompilerParams(
            use_tc_tiling_on_sc=False, needs_layout_passes=False),
    )
    def _sc_gather_wide(idx_hbm, cs2d_hbm, wsum_out,
                        idx_v, widx_v, wrows, wv, sem_w):
        wid = lax.axis_index("s") * 2 + lax.axis_index("c")
        base = wid * IDX_PER_TILE
        iota16 = lax.iota(jnp.int32, 16)

        def chunk_body(ci, carry):
            off = base + ci * CHUNK
            pltpu.sync_copy(idx_hbm.at[pl.ds(off, CHUNK)], idx_v)

            def wi_body(t, c2):
                iv = idx_v[pl.ds(t * 16, 16)]
                widx_v[pl.ds(t * 16, 16)] = lax.shift_right_logical(iv, 4)
                return c2
            lax.fori_loop(0, CHUNK // 16, wi_body, 0)

            pltpu.async_copy(cs2d_hbm.at[widx_v], wrows, sem_w).wait()

            def ex_body(t, c2):
                rows = iota16 + t * 16
                cm = lax.bitwise_and(idx_v[pl.ds(t * 16, 16)], 15)
                wv[pl.ds(t * 16, 16)] = plsc.load_gather(wrows, [rows, cm])
                return c2
            lax.fori_loop(0, CHUNK // 16, ex_body, 0)
            pltpu.sync_copy(wv, wsum_out.at[pl.ds(off, CHUNK)])
            return carry

        lax.fori_loop(0, NCHUNK, chunk_body, 0)

    return _sc_gather_wide


def _colsum_body(wt_ref, out_ref):
    s = jnp.sum(wt_ref[...], axis=0)
    out_ref[...] = s.reshape(out_ref.shape)


_CS_BLK = 32768


def _stats_body(x_ref, s1_ref, s2_ref):
    i = pl.program_id(0)

    @pl.when(i == 0)
    def _():
        s1_ref[...] = jnp.zeros_like(s1_ref)
        s2_ref[...] = jnp.zeros_like(s2_ref)

    xb = x_ref[...]
    s1_ref[...] += jnp.sum(xb, axis=0, keepdims=True)
    s2_ref[...] += jnp.sum(xb * xb, axis=0, keepdims=True)


def _mlp_body(s1_ref, s2_ref, gamma_ref, beta_ref, w1_ref, b1_ref,
              w2_ref, b2_ref, w3_ref, b3_ref, x_ref, wide_ref, out_ref):
    mean = s1_ref[...] * (1.0 / B)
    var = s2_ref[...] * (1.0 / B) - mean * mean
    scale = gamma_ref[...] * lax.rsqrt(var + 1e-5)
    shift = beta_ref[...] - mean * scale
    xn = x_ref[...] * scale + shift
    h = jnp.dot(xn.astype(jnp.bfloat16), w1_ref[...],
                preferred_element_type=jnp.float32)
    h = jnp.maximum(h + b1_ref[...], 0.0)
    h = jnp.dot(h.astype(jnp.bfloat16), w2_ref[...],
                preferred_element_type=jnp.float32)
    h = jnp.maximum(h + b2_ref[...], 0.0)
    d = jnp.dot(h.astype(jnp.bfloat16), w3_ref[...],
                preferred_element_type=jnp.float32)
    d = d + b3_ref[...]
    w = jnp.sum(wide_ref[...], axis=1, keepdims=True)
    out_ref[...] = jax.nn.sigmoid(d + w)


_STATS_BLK = 2048
_MLP_BLK = 2048


@jax.jit
def kernel(inputs, wide_table, deep_table, gamma, beta, W1, b1, W2, b2, W3, b3):
    idx_flat = inputs.reshape(B * F).astype(jnp.int32)

    tail_deep = deep_table[REM0 + REM_A:, :].reshape(REM_B * E)
    deep_dense = _make_sc_format()(deep_table.T, tail_deep)
    ncs = (V + _CS_BLK - 1) // _CS_BLK
    wide_colsum = pl.pallas_call(
        _colsum_body,
        grid=(ncs,),
        in_specs=[pl.BlockSpec((E, _CS_BLK), lambda i: (0, i))],
        out_specs=pl.BlockSpec((_CS_BLK // 128, 128), lambda i: (i, 0)),
        out_shape=jax.ShapeDtypeStruct((8192, 128), jnp.float32),
    )(wide_table.T)
    deep_rows = _make_sc_gather_deep()(idx_flat, deep_dense.reshape(V, E))
    wsum = _make_sc_gather_wide()(idx_flat, wide_colsum.reshape(65536, E))
    x = deep_rows.reshape(B, D)
    wmat = wsum.reshape(B, F)

    s1, s2 = pl.pallas_call(
        _stats_body,
        grid=(B // _STATS_BLK,),
        in_specs=[pl.BlockSpec((_STATS_BLK, D), lambda i: (i, 0))],
        out_specs=[
            pl.BlockSpec((1, D), lambda i: (0, 0)),
            pl.BlockSpec((1, D), lambda i: (0, 0)),
        ],
        out_shape=[
            jax.ShapeDtypeStruct((1, D), jnp.float32),
            jax.ShapeDtypeStruct((1, D), jnp.float32),
        ],
    )(x)

    out = pl.pallas_call(
        _mlp_body,
        grid=(B // _MLP_BLK,),
        in_specs=[
            pl.BlockSpec((1, D), lambda i: (0, 0)),
            pl.BlockSpec((1, D), lambda i: (0, 0)),
            pl.BlockSpec((1, D), lambda i: (0, 0)),
            pl.BlockSpec((1, D), lambda i: (0, 0)),
            pl.BlockSpec((D, H1), lambda i: (0, 0)),
            pl.BlockSpec((1, H1), lambda i: (0, 0)),
            pl.BlockSpec((H1, H2), lambda i: (0, 0)),
            pl.BlockSpec((1, H2), lambda i: (0, 0)),
            pl.BlockSpec((H2, 1), lambda i: (0, 0)),
            pl.BlockSpec((1, 1), lambda i: (0, 0)),
            pl.BlockSpec((_MLP_BLK, D), lambda i: (i, 0)),
            pl.BlockSpec((_MLP_BLK, F), lambda i: (i, 0)),
        ],
        out_specs=pl.BlockSpec((_MLP_BLK, 1), lambda i: (i, 0)),
        out_shape=jax.ShapeDtypeStruct((B, 1), jnp.float32),
    )(
        s1, s2,
        gamma.reshape(1, D), beta.reshape(1, D),
        W1.astype(jnp.bfloat16), b1.reshape(1, H1),
        W2.astype(jnp.bfloat16), b2.reshape(1, H2),
        W3.astype(jnp.bfloat16), b3.reshape(1, 1),
        x, wmat,
    )
    return out


# speedup vs baseline: 1.1661x; 1.0329x over previous
import functools

import jax
import jax.numpy as jnp
from jax import lax
from jax.experimental import pallas as pl
from jax.experimental.pallas import tpu as pltpu
from jax.experimental.pallas import tpu_sc as plsc

B = 16384
F = 26
V = 1000000
E = 16
D = F * E
H1 = 1024
H2 = 512

NUM_WORKERS = 32

SLAB = 512
NSLAB = 61 * NUM_WORKERS
SLAB_REM = V - NSLAB * SLAB
REM0 = NSLAB * SLAB
REM_A = 512
REM_B = SLAB_REM - REM_A
KPT = NSLAB // NUM_WORKERS

IDX_PER_TILE = (B * F) // NUM_WORKERS
CHUNK = 3328
NCHUNK = IDX_PER_TILE // CHUNK
DCHUNK = 6656
NDCHUNK = IDX_PER_TILE // DCHUNK


def _transpose_slab(tbuf, obuf, iota16, n_ch):
    def ch_body(ch, c):
        for e in range(E):
            vals = tbuf[e, pl.ds(ch * 16, 16)]
            idxv = iota16 * 16 + (ch * 256 + e)
            plsc.store_scatter(obuf, [idxv], vals)
        return c
    lax.fori_loop(0, n_ch, ch_body, 0)


def _colsum_slab(tbuf, csbuf, n_ch):
    def ch_body(ch, c):
        acc = tbuf[0, pl.ds(ch * 16, 16)]
        for e in range(1, E):
            acc = acc + tbuf[e, pl.ds(ch * 16, 16)]
        csbuf[pl.ds(ch * 16, 16)] = acc
        return c
    lax.fori_loop(0, n_ch, ch_body, 0)


@functools.lru_cache(maxsize=1)
def _make_sc_format():
    mesh = plsc.VectorSubcoreMesh(core_axis_name="c", subcore_axis_name="s")

    @functools.partial(
        pl.kernel,
        mesh=mesh,
        out_type=jax.ShapeDtypeStruct((V * E,), jnp.float32),
        scratch_types=[
            pltpu.VMEM((E, SLAB), jnp.float32),
            pltpu.VMEM((E, SLAB), jnp.float32),
            pltpu.VMEM((SLAB * E,), jnp.float32),
            pltpu.VMEM((SLAB * E,), jnp.float32),
            pltpu.SemaphoreType.DMA,
            pltpu.SemaphoreType.DMA,
            pltpu.SemaphoreType.DMA,
            pltpu.SemaphoreType.DMA,
        ],
        compiler_params=pltpu.CompilerParams(
            use_tc_tiling_on_sc=True, needs_layout_passes=False),
    )
    def _sc_format(deepT, tail_deep, ddense,
                   tbuf0, tbuf1, obuf0, obuf1,
                   sin0, sin1, sout0, sout1):
        wid = lax.axis_index("s") * 2 + lax.axis_index("c")
        iota16 = lax.iota(jnp.int32, 16)
        tb = (tbuf0, tbuf1)
        ob = (obuf0, obuf1)
        sin = (sin0, sin1)
        sout = (sout0, sout1)

        def col0(k):
            return (wid + NUM_WORKERS * k) * SLAB

        def wait_in(p):
            pltpu.make_async_copy(
                deepT.at[:, pl.ds(0, SLAB)], tb[p], sin[p]).wait()

        def wait_out(p):
            pltpu.make_async_copy(
                ob[p], ddense.at[pl.ds(0, SLAB * E)], sout[p]).wait()

        def compute_and_out(k, p):
            c0 = col0(k)
            _transpose_slab(tb[p], ob[p], iota16, SLAB // 16)
            pltpu.async_copy(
                ob[p], ddense.at[pl.ds(c0 * E, SLAB * E)], sout[p])

        for p in range(2):
            pltpu.async_copy(deepT.at[:, pl.ds(col0(p), SLAB)],
                             tb[p], sin[p])

        def body2(k2, c):
            for p in range(2):
                k = 2 * k2 + p
                wait_in(p)

                @pl.when(k2 >= 1)
                def _():
                    wait_out(p)
                compute_and_out(k, p)
                if p == 0:
                    pltpu.async_copy(
                        deepT.at[:, pl.ds(col0(k + 2), SLAB)], tb[p], sin[p])
                else:
                    @pl.when(k2 < 29)
                    def _():
                        pltpu.async_copy(
                            deepT.at[:, pl.ds(col0(k + 2), SLAB)],
                            tb[p], sin[p])
            return c

        lax.fori_loop(0, 30, body2, 0)
        wait_in(0)
        wait_out(0)
        compute_and_out(60, 0)
        wait_out(0)
        wait_out(1)

        @pl.when(wid == 28)
        def _():
            pltpu.sync_copy(deepT.at[:, pl.ds(REM0, REM_A)],
                            tbuf0.at[:, pl.ds(0, REM_A)])
            _transpose_slab(tbuf0, obuf0, iota16, REM_A // 16)
            pltpu.sync_copy(obuf0.at[pl.ds(0, REM_A * E)],
                            ddense.at[pl.ds(REM0 * E, REM_A * E)])

        @pl.when(wid == 29)
        def _():
            pltpu.sync_copy(tail_deep, obuf1.at[pl.ds(0, REM_B * E)])
            pltpu.sync_copy(obuf1.at[pl.ds(0, REM_B * E)],
                            ddense.at[pl.ds((REM0 + REM_A) * E, REM_B * E)])

    return _sc_format


@functools.lru_cache(maxsize=1)
def _make_sc_gather_deep():
    mesh = plsc.VectorSubcoreMesh(core_axis_name="c", subcore_axis_name="s")

    @functools.partial(
        pl.kernel,
        mesh=mesh,
        out_type=jax.ShapeDtypeStruct((B * F, E), jnp.float32),
        scratch_types=[
            pltpu.VMEM((DCHUNK,), jnp.int32),
            pltpu.VMEM((DCHUNK, E), jnp.float32),
            pltpu.SemaphoreType.DMA,
            pltpu.SemaphoreType.DMA,
        ],
        compiler_params=pltpu.CompilerParams(
            use_tc_tiling_on_sc=False, needs_layout_passes=False),
    )
    def _sc_gather_deep(idx_hbm, deep_hbm, deep_out, idx_v, drows,
                        sem_g, sem_o):
        wid = lax.axis_index("s") * 2 + lax.axis_index("c")
        base = wid * IDX_PER_TILE
        pltpu.sync_copy(idx_hbm.at[pl.ds(base, DCHUNK)], idx_v)
        pltpu.async_copy(deep_hbm.at[idx_v], drows, sem_g).wait()
        pltpu.async_copy(drows, deep_out.at[pl.ds(base, DCHUNK)], sem_o)
        pltpu.sync_copy(idx_hbm.at[pl.ds(base + DCHUNK, DCHUNK)], idx_v)
        pltpu.make_async_copy(
            drows, deep_out.at[pl.ds(0, DCHUNK)], sem_o).wait()
        pltpu.async_copy(deep_hbm.at[idx_v], drows, sem_g).wait()
        pltpu.sync_copy(drows, deep_out.at[pl.ds(base + DCHUNK, DCHUNK)])

    return _sc_gather_deep


@functools.lru_cache(maxsize=1)
def _make_sc_gather_wide():
    mesh = plsc.VectorSubcoreMesh(core_axis_name="c", subcore_axis_name="s")

    @functools.partial(
        pl.kernel,
        mesh=mesh,
        out_type=jax.ShapeDtypeStruct((B * F,), jnp.float32),
        scratch_types=[
            pltpu.VMEM((CHUNK,), jnp.int32),
            pltpu.VMEM((CHUNK,), jnp.int32),
            pltpu.VMEM((CHUNK,), jnp.int32),
            pltpu.VMEM((CHUNK,), jnp.int32),
            pltpu.VMEM((CHUNK, E), jnp.float32),
            pltpu.VMEM((CHUNK, E), jnp.float32),
            pltpu.VMEM((CHUNK,), jnp.float32),
            pltpu.SemaphoreType.DMA,
            pltpu.SemaphoreType.DMA,
        ],
        compiler_params=pltpu.CompilerParams(
            use_tc_tiling_on_sc=False, needs_layout_passes=False),
    )
    def _sc_gather_wide(idx_hbm, cs2d_hbm, wsum_out,
                        idx_v0, idx_v1, widx_v0, widx_v1,
                        wrows0, wrows1, wv, sem0, sem1):
        wid = lax.axis_index("s") * 2 + lax.axis_index("c")
        base = wid * IDX_PER_TILE
        iota16 = lax.iota(jnp.int32, 16)
        iv_ = (idx_v0, idx_v1)
        wi_ = (widx_v0, widx_v1)
        wr_ = (wrows0, wrows1)
        sm_ = (sem0, sem1)

        def start_gather(ci, p):
            off = base + ci * CHUNK
            pltpu.sync_copy(idx_hbm.at[pl.ds(off, CHUNK)], iv_[p])

            def wi_body(t, c2):
                v = iv_[p][pl.ds(t * 16, 16)]
                wi_[p][pl.ds(t * 16, 16)] = lax.shift_right_logical(v, 4)
                return c2
            lax.fori_loop(0, CHUNK // 16, wi_body, 0)
            pltpu.async_copy(cs2d_hbm.at[wi_[p]], wr_[p], sm_[p])

        def finish(ci, p):
            off = base + ci * CHUNK
            pltpu.make_async_copy(
                cs2d_hbm.at[wi_[p]], wr_[p], sm_[p]).wait()

            def ex_body(t, c2):
                rows = iota16 + t * 16
                cm = lax.bitwise_and(iv_[p][pl.ds(t * 16, 16)], 15)
                wv[pl.ds(t * 16, 16)] = plsc.load_gather(wr_[p], [rows, cm])
                return c2
            lax.fori_loop(0, CHUNK // 16, ex_body, 0)
            pltpu.sync_copy(wv, wsum_out.at[pl.ds(off, CHUNK)])

        start_gather(0, 0)
        for ci in range(NCHUNK):
            p = ci & 1
            if ci + 1 < NCHUNK:
                start_gather(ci + 1, 1 - p)
            finish(ci, p)

    return _sc_gather_wide


def _colsum_body(wt_ref, out_ref):
    s = jnp.sum(wt_ref[...], axis=0)
    out_ref[...] = s.reshape(out_ref.shape)


_CS_BLK = 32768


def _stats_body(x_ref, s1_ref, s2_ref):
    i = pl.program_id(0)

    @pl.when(i == 0)
    def _():
        s1_ref[...] = jnp.zeros_like(s1_ref)
        s2_ref[...] = jnp.zeros_like(s2_ref)

    xb = x_ref[...]
    s1_ref[...] += jnp.sum(xb, axis=0, keepdims=True)
    s2_ref[...] += jnp.sum(xb * xb, axis=0, keepdims=True)


def _mlp_body(s1_ref, s2_ref, gamma_ref, beta_ref, w1_ref, b1_ref,
              w2_ref, b2_ref, w3_ref, b3_ref, x_ref, wide_ref, out_ref):
    mean = s1_ref[...] * (1.0 / B)
    var = s2_ref[...] * (1.0 / B) - mean * mean
    scale = gamma_ref[...] * lax.rsqrt(var + 1e-5)
    shift = beta_ref[...] - mean * scale
    xn = x_ref[...] * scale + shift
    h = jnp.dot(xn.astype(jnp.bfloat16), w1_ref[...],
                preferred_element_type=jnp.float32)
    h = jnp.maximum(h + b1_ref[...], 0.0)
    h = jnp.dot(h.astype(jnp.bfloat16), w2_ref[...],
                preferred_element_type=jnp.float32)
    h = jnp.maximum(h + b2_ref[...], 0.0)
    d = jnp.dot(h.astype(jnp.bfloat16), w3_ref[...],
                preferred_element_type=jnp.float32)
    d = d + b3_ref[...]
    w = jnp.sum(wide_ref[...], axis=1, keepdims=True)
    out_ref[...] = jax.nn.sigmoid(d + w)


_STATS_BLK = 2048
_MLP_BLK = 2048


@jax.jit
def kernel(inputs, wide_table, deep_table, gamma, beta, W1, b1, W2, b2, W3, b3):
    idx_flat = inputs.reshape(B * F).astype(jnp.int32)

    tail_deep = deep_table[REM0 + REM_A:, :].reshape(REM_B * E)
    deep_dense = _make_sc_format()(deep_table.T, tail_deep)
    ncs = (V + _CS_BLK - 1) // _CS_BLK
    wide_colsum = pl.pallas_call(
        _colsum_body,
        grid=(ncs,),
        in_specs=[pl.BlockSpec((E, _CS_BLK), lambda i: (0, i))],
        out_specs=pl.BlockSpec((_CS_BLK // 128, 128), lambda i: (i, 0)),
        out_shape=jax.ShapeDtypeStruct((8192, 128), jnp.float32),
    )(wide_table.T)
    deep_rows = _make_sc_gather_deep()(idx_flat, deep_dense.reshape(V, E))
    wsum = _make_sc_gather_wide()(idx_flat, wide_colsum.reshape(65536, E))
    x = deep_rows.reshape(B, D)
    wmat = wsum.reshape(B, F)

    s1, s2 = pl.pallas_call(
        _stats_body,
        grid=(B // _STATS_BLK,),
        in_specs=[pl.BlockSpec((_STATS_BLK, D), lambda i: (i, 0))],
        out_specs=[
            pl.BlockSpec((1, D), lambda i: (0, 0)),
            pl.BlockSpec((1, D), lambda i: (0, 0)),
        ],
        out_shape=[
            jax.ShapeDtypeStruct((1, D), jnp.float32),
            jax.ShapeDtypeStruct((1, D), jnp.float32),
        ],
    )(x)

    out = pl.pallas_call(
        _mlp_body,
        grid=(B // _MLP_BLK,),
        in_specs=[
            pl.BlockSpec((1, D), lambda i: (0, 0)),
            pl.BlockSpec((1, D), lambda i: (0, 0)),
            pl.BlockSpec((1, D), lambda i: (0, 0)),
            pl.BlockSpec((1, D), lambda i: (0, 0)),
            pl.BlockSpec((D, H1), lambda i: (0, 0)),
            pl.BlockSpec((1, H1), lambda i: (0, 0)),
            pl.BlockSpec((H1, H2), lambda i: (0, 0)),
            pl.BlockSpec((1, H2), lambda i: (0, 0)),
            pl.BlockSpec((H2, 1), lambda i: (0, 0)),
            pl.BlockSpec((1, 1), lambda i: (0, 0)),
            pl.BlockSpec((_MLP_BLK, D), lambda i: (i, 0)),
            pl.BlockSpec((_MLP_BLK, F), lambda i: (i, 0)),
        ],
        out_specs=pl.BlockSpec((_MLP_BLK, 1), lambda i: (i, 0)),
        out_shape=jax.ShapeDtypeStruct((B, 1), jnp.float32),
    )(
        s1, s2,
        gamma.reshape(1, D), beta.reshape(1, D),
        W1.astype(jnp.bfloat16), b1.reshape(1, H1),
        W2.astype(jnp.bfloat16), b2.reshape(1, H2),
        W3.astype(jnp.bfloat16), b3.reshape(1, 1),
        x, wmat,
    )
    return out
